# baseline (speedup 1.0000x reference)
import jax
import jax.numpy as jnp
from jax.experimental import pallas as pl
from jax.experimental.pallas import tpu as pltpu

import numpy as np
from jax.sharding import Mesh, PartitionSpec as P

NNZ = 16384
OUT_F = 1024
IN_F = 1024

NCHUNK = 8
KC = NNZ // NCHUNK
NB = 2
CB = IN_F // NB

BT = 1024


def _scatter_kernel(idx_ref, val_ref, base_ref, alpha_ref, w_ref, acc_ref):
    b = pl.program_id(0)
    k = pl.program_id(1)

    @pl.when(k == 0)
    def _():
        acc_ref[...] = jnp.zeros_like(acc_ref)

    idx = idx_ref[0]
    rows = jax.lax.shift_right_logical(idx, 10)
    cols = jnp.bitwise_and(idx, IN_F - 1) - b * CB
    vals = val_ref[0]

    iota_r = jax.lax.broadcasted_iota(jnp.int32, (OUT_F, KC), 0)
    iota_c = jax.lax.broadcasted_iota(jnp.int32, (CB, KC), 0)

    rt = jnp.where(jnp.broadcast_to(rows, (OUT_F, KC)) == iota_r, 1.0, 0.0
                   ).astype(jnp.bfloat16)
    ct = jnp.where(jnp.broadcast_to(cols, (CB, KC)) == iota_c,
                   jnp.broadcast_to(vals, (CB, KC)), 0.0).astype(jnp.bfloat16)

    acc_ref[...] += jax.lax.dot_general(
        rt, ct, (((1,), (1,)), ((), ())), preferred_element_type=jnp.float32)

    @pl.when(k == NCHUNK - 1)
    def _():
        w_ref[...] = (base_ref[...] + alpha_ref[0, 0] * acc_ref[...]
                      ).astype(jnp.bfloat16)


def _matmul_kernel(x_ref, w_ref, o_ref):
    xb = x_ref[...].astype(jnp.bfloat16)
    o_ref[...] = jax.lax.dot_general(
        xb, w_ref[...], (((1,), (1,)), ((), ())),
        preferred_element_type=jnp.float32)


def _build_w_eff(idx3, vals3, base32, alpha2d, *, interpret=False):
    return pl.pallas_call(
        _scatter_kernel,
        grid=(NB, NCHUNK),
        in_specs=[
            pl.BlockSpec((1, 1, KC), lambda b, k: (k, 0, 0)),
            pl.BlockSpec((1, 1, KC), lambda b, k: (k, 0, 0)),
            pl.BlockSpec((OUT_F, CB), lambda b, k: (0, b)),
            pl.BlockSpec(memory_space=pltpu.SMEM),
        ],
        out_specs=pl.BlockSpec((OUT_F, CB), lambda b, k: (0, b)),
        out_shape=jax.ShapeDtypeStruct((OUT_F, IN_F), jnp.bfloat16),
        scratch_shapes=[pltpu.VMEM((OUT_F, CB), jnp.float32)],
        compiler_params=pltpu.CompilerParams(
            dimension_semantics=("arbitrary", "arbitrary"),
        ),
        name="ortho_scatter_weff",
        interpret=interpret,
    )(idx3, vals3, base32, alpha2d)


def _apply(xf, w_eff, *, interpret=False):
    t = xf.shape[0]
    return pl.pallas_call(
        _matmul_kernel,
        grid=(t // BT,),
        in_specs=[
            pl.BlockSpec((BT, IN_F), lambda i: (i, 0)),
            pl.BlockSpec((OUT_F, IN_F), lambda i: (0, 0)),
        ],
        out_specs=pl.BlockSpec((BT, OUT_F), lambda i: (i, 0)),
        out_shape=jax.ShapeDtypeStruct((t, OUT_F), jnp.float32),
        compiler_params=pltpu.CompilerParams(
            dimension_semantics=("arbitrary",),
        ),
        name="ortho_linear_matmul",
        interpret=interpret,
    )(xf, w_eff)


def kernel(x, base_weight, ortho_values, ortho_indices, alpha, *, interpret=False):
    out_f, in_f = base_weight.shape
    lead = x.shape[:-1]
    xf = x.reshape(-1, in_f)

    idx3 = ortho_indices.reshape(NCHUNK, 1, KC)
    vals3 = ortho_values.astype(jnp.float32).reshape(NCHUNK, 1, KC)
    base32 = base_weight.astype(jnp.float32)
    alpha2d = alpha.astype(jnp.float32).reshape(1, 1)

    ndev = min(2, jax.device_count())
    mesh = Mesh(np.asarray(jax.devices()[:ndev]), ("d",))

    def _shard_fn(xs, idx_s, vals_s, base_s, alpha_s):
        w_eff = _build_w_eff(idx_s, vals_s, base_s, alpha_s,
                             interpret=interpret)
        return _apply(xs, w_eff, interpret=interpret)

    out = jax.shard_map(
        _shard_fn, mesh=mesh,
        in_specs=(P("d", None), P(None, None, None), P(None, None, None),
                  P(None, None), P(None, None)),
        out_specs=P("d", None), check_vma=False,
    )(xf, idx3, vals3, base32, alpha2d)
    return out.reshape(*lead, out_f)


# speedup vs baseline: 4.2881x; 4.2881x over previous
import jax
import jax.numpy as jnp
from jax.experimental import pallas as pl
from jax.experimental.pallas import tpu as pltpu

NNZ = 16384
OUT_F = 1024
IN_F = 1024

NCHUNK = 8
KC = NNZ // NCHUNK

BT = 1024


def _fused_kernel(idx_ref, val_ref, base_ref, alpha_ref, x_ref, o_ref, w_ref):
    i = pl.program_id(0)

    @pl.when(i == 0)
    def _():
        alpha = alpha_ref[0, 0]
        acc = None
        for k in range(NCHUNK):
            sl = slice(k * KC, (k + 1) * KC)
            idx = idx_ref[:, sl]
            rows = jax.lax.shift_right_logical(idx, 10)
            cols = jnp.bitwise_and(idx, IN_F - 1)
            vals = val_ref[:, sl]
            iota = jax.lax.broadcasted_iota(jnp.int32, (OUT_F, KC), 0)
            rt = jnp.where(jnp.broadcast_to(rows, (OUT_F, KC)) == iota,
                           1.0, 0.0).astype(jnp.bfloat16)
            ct = jnp.where(jnp.broadcast_to(cols, (IN_F, KC)) == iota,
                           jnp.broadcast_to(vals, (IN_F, KC)),
                           0.0).astype(jnp.bfloat16)
            d = jax.lax.dot_general(
                rt, ct, (((1,), (1,)), ((), ())),
                preferred_element_type=jnp.float32)
            acc = d if acc is None else acc + d
        w_ref[...] = (base_ref[...] + alpha * acc).astype(jnp.bfloat16)

    xb = x_ref[...].astype(jnp.bfloat16)
    o_ref[...] = jax.lax.dot_general(
        xb, w_ref[...], (((1,), (1,)), ((), ())),
        preferred_element_type=jnp.float32)


def _run(xf, idx2, vals2, base32, alpha2d, *, interpret=False):
    t = xf.shape[0]
    return pl.pallas_call(
        _fused_kernel,
        grid=(t // BT,),
        in_specs=[
            pl.BlockSpec((1, NNZ), lambda i: (0, 0)),
            pl.BlockSpec((1, NNZ), lambda i: (0, 0)),
            pl.BlockSpec((OUT_F, IN_F), lambda i: (0, 0)),
            pl.BlockSpec(memory_space=pltpu.SMEM),
            pl.BlockSpec((BT, IN_F), lambda i: (i, 0)),
        ],
        out_specs=pl.BlockSpec((BT, OUT_F), lambda i: (i, 0)),
        out_shape=jax.ShapeDtypeStruct((t, OUT_F), jnp.float32),
        scratch_shapes=[pltpu.VMEM((OUT_F, IN_F), jnp.bfloat16)],
        compiler_params=pltpu.CompilerParams(
            dimension_semantics=("arbitrary",),
            vmem_limit_bytes=56 * 1024 * 1024,
        ),
        name="ortho_linear_fused",
        interpret=interpret,
    )(idx2, vals2, base32, alpha2d, xf)


def kernel(x, base_weight, ortho_values, ortho_indices, alpha, *, interpret=False):
    out_f, in_f = base_weight.shape
    lead = x.shape[:-1]
    xf = x.reshape(-1, in_f)

    idx2 = ortho_indices.reshape(1, NNZ)
    vals2 = ortho_values.astype(jnp.float32).reshape(1, NNZ)
    base32 = base_weight.astype(jnp.float32)
    alpha2d = alpha.astype(jnp.float32).reshape(1, 1)

    out = _run(xf, idx2, vals2, base32, alpha2d, interpret=interpret)
    return out.reshape(*lead, out_f)


# speedup vs baseline: 4.5682x; 1.0653x over previous
import jax
import jax.numpy as jnp
from jax.experimental import pallas as pl
from jax.experimental.pallas import tpu as pltpu

NNZ = 16384
OUT_F = 1024
IN_F = 1024

NCHUNK = 8
KC = NNZ // NCHUNK

BT = 2048


def _fused_kernel(idx_ref, val_ref, base_ref, alpha_ref, x_ref, o_ref, w_ref):
    i = pl.program_id(0)

    @pl.when(i == 0)
    def _():
        alpha = alpha_ref[0, 0]
        acc = None
        for k in range(NCHUNK):
            sl = slice(k * KC, (k + 1) * KC)
            idx = idx_ref[:, sl]
            rows = jax.lax.shift_right_logical(idx, 10)
            cols = jnp.bitwise_and(idx, IN_F - 1)
            vals = val_ref[:, sl]
            iota = jax.lax.broadcasted_iota(jnp.int32, (OUT_F, KC), 0)
            rt = jnp.where(jnp.broadcast_to(rows, (OUT_F, KC)) == iota,
                           1.0, 0.0).astype(jnp.bfloat16)
            ct = jnp.where(jnp.broadcast_to(cols, (IN_F, KC)) == iota,
                           jnp.broadcast_to(vals, (IN_F, KC)),
                           0.0).astype(jnp.bfloat16)
            d = jax.lax.dot_general(
                rt, ct, (((1,), (1,)), ((), ())),
                preferred_element_type=jnp.float32)
            acc = d if acc is None else acc + d
        w_ref[...] = (base_ref[...] + alpha * acc).astype(jnp.bfloat16)

    xb = x_ref[...].astype(jnp.bfloat16)
    o_ref[...] = jax.lax.dot_general(
        xb, w_ref[...], (((1,), (1,)), ((), ())),
        preferred_element_type=jnp.float32)


def _run(xf, idx2, vals2, base32, alpha2d, *, interpret=False):
    t = xf.shape[0]
    return pl.pallas_call(
        _fused_kernel,
        grid=(t // BT,),
        in_specs=[
            pl.BlockSpec((1, NNZ), lambda i: (0, 0)),
            pl.BlockSpec((1, NNZ), lambda i: (0, 0)),
            pl.BlockSpec((OUT_F, IN_F), lambda i: (0, 0)),
            pl.BlockSpec(memory_space=pltpu.SMEM),
            pl.BlockSpec((BT, IN_F), lambda i: (i, 0)),
        ],
        out_specs=pl.BlockSpec((BT, OUT_F), lambda i: (i, 0)),
        out_shape=jax.ShapeDtypeStruct((t, OUT_F), jnp.float32),
        scratch_shapes=[pltpu.VMEM((OUT_F, IN_F), jnp.bfloat16)],
        compiler_params=pltpu.CompilerParams(
            dimension_semantics=("arbitrary",),
            vmem_limit_bytes=56 * 1024 * 1024,
        ),
        name="ortho_linear_fused",
        interpret=interpret,
    )(idx2, vals2, base32, alpha2d, xf)


def kernel(x, base_weight, ortho_values, ortho_indices, alpha, *, interpret=False):
    out_f, in_f = base_weight.shape
    lead = x.shape[:-1]
    xf = x.reshape(-1, in_f)

    idx2 = ortho_indices.reshape(1, NNZ)
    vals2 = ortho_values.astype(jnp.float32).reshape(1, NNZ)
    base32 = base_weight.astype(jnp.float32)
    alpha2d = alpha.astype(jnp.float32).reshape(1, 1)

    out = _run(xf, idx2, vals2, base32, alpha2d, interpret=interpret)
    return out.reshape(*lead, out_f)


# speedup vs baseline: 4.9453x; 1.0826x over previous
import jax
import jax.numpy as jnp
from jax.experimental import pallas as pl
from jax.experimental.pallas import tpu as pltpu

NNZ = 16384
OUT_F = 1024
IN_F = 1024

NCHUNK = 8
KC = NNZ // NCHUNK

BT = 1024
DEPTH = 6


def _fused_kernel(idx_ref, val_ref, base_ref, alpha_ref, x_hbm,
                  o_ref, w_ref, xbufs, sems):
    i = pl.program_id(0)
    nt = pl.num_programs(0)

    @pl.when(i == 0)
    def _():
        for d in range(DEPTH):
            pltpu.make_async_copy(
                x_hbm.at[pl.ds(d * BT, BT), :], xbufs.at[d], sems.at[d]
            ).start()
        alpha = alpha_ref[0, 0]
        acc = None
        for k in range(NCHUNK):
            sl = slice(k * KC, (k + 1) * KC)
            idx = idx_ref[:, sl]
            rows = jax.lax.shift_right_logical(idx, 10)
            cols = jnp.bitwise_and(idx, IN_F - 1)
            vals = val_ref[:, sl]
            iota = jax.lax.broadcasted_iota(jnp.int32, (OUT_F, KC), 0)
            rt = jnp.where(jnp.broadcast_to(rows, (OUT_F, KC)) == iota,
                           1.0, 0.0).astype(jnp.bfloat16)
            ct = jnp.where(jnp.broadcast_to(cols, (IN_F, KC)) == iota,
                           jnp.broadcast_to(vals, (IN_F, KC)),
                           0.0).astype(jnp.bfloat16)
            d = jax.lax.dot_general(
                rt, ct, (((1,), (1,)), ((), ())),
                preferred_element_type=jnp.float32)
            acc = d if acc is None else acc + d
        w_ref[...] = (base_ref[...] + alpha * acc).astype(jnp.bfloat16)

    slot = jax.lax.rem(i, DEPTH)
    pltpu.make_async_copy(xbufs.at[slot], xbufs.at[slot], sems.at[slot]).wait()
    xb = xbufs[slot].astype(jnp.bfloat16)
    o_ref[...] = jax.lax.dot_general(
        xb, w_ref[...], (((1,), (1,)), ((), ())),
        preferred_element_type=jnp.float32)

    @pl.when(i + DEPTH < nt)
    def _():
        nxt = pl.multiple_of((i + DEPTH) * BT, BT)
        pltpu.make_async_copy(
            x_hbm.at[pl.ds(nxt, BT), :], xbufs.at[slot], sems.at[slot]
        ).start()


def _run(xf, idx2, vals2, base32, alpha2d, *, interpret=False):
    t = xf.shape[0]
    return pl.pallas_call(
        _fused_kernel,
        grid=(t // BT,),
        in_specs=[
            pl.BlockSpec((1, NNZ), lambda i: (0, 0)),
            pl.BlockSpec((1, NNZ), lambda i: (0, 0)),
            pl.BlockSpec((OUT_F, IN_F), lambda i: (0, 0)),
            pl.BlockSpec(memory_space=pltpu.SMEM),
            pl.BlockSpec(memory_space=pl.ANY),
        ],
        out_specs=pl.BlockSpec((BT, OUT_F), lambda i: (i, 0)),
        out_shape=jax.ShapeDtypeStruct((t, OUT_F), jnp.float32),
        scratch_shapes=[
            pltpu.VMEM((OUT_F, IN_F), jnp.bfloat16),
            pltpu.VMEM((DEPTH, BT, IN_F), jnp.float32),
            pltpu.SemaphoreType.DMA((DEPTH,)),
        ],
        compiler_params=pltpu.CompilerParams(
            dimension_semantics=("arbitrary",),
            vmem_limit_bytes=56 * 1024 * 1024,
        ),
        name="ortho_linear_fused",
        interpret=interpret,
    )(idx2, vals2, base32, alpha2d, xf)


def kernel(x, base_weight, ortho_values, ortho_indices, alpha, *, interpret=False):
    out_f, in_f = base_weight.shape
    lead = x.shape[:-1]
    xf = x.reshape(-1, in_f)

    idx2 = ortho_indices.reshape(1, NNZ)
    vals2 = ortho_values.astype(jnp.float32).reshape(1, NNZ)
    base32 = base_weight.astype(jnp.float32)
    alpha2d = alpha.astype(jnp.float32).reshape(1, 1)

    out = _run(xf, idx2, vals2, base32, alpha2d, interpret=interpret)
    return out.reshape(*lead, out_f)


# speedup vs baseline: 5.0018x; 1.0114x over previous
import jax
import jax.numpy as jnp
from jax.experimental import pallas as pl
from jax.experimental.pallas import tpu as pltpu

NNZ = 16384
OUT_F = 1024
IN_F = 1024

NCHUNK = 8
KC = NNZ // NCHUNK

BT = 1024
DEPTH = 6


def _fused_kernel(idx_ref, val_ref, base_ref, x_hbm, o_ref, w_ref, xbufs, sems):
    i = pl.program_id(0)
    nt = pl.num_programs(0)

    @pl.when(i == 0)
    def _():
        for d in range(DEPTH):
            pltpu.make_async_copy(
                x_hbm.at[pl.ds(d * BT, BT), :], xbufs.at[d], sems.at[d]
            ).start()
        one = jnp.bfloat16(1.0)
        zero = jnp.bfloat16(0.0)
        acc = None
        for k in range(NCHUNK):
            sl = slice(k * KC, (k + 1) * KC)
            idx = idx_ref[:, sl]
            rows = jax.lax.shift_right_logical(idx, 10).astype(jnp.int16)
            cols = jnp.bitwise_and(idx, IN_F - 1).astype(jnp.int16)
            vals = val_ref[:, sl]
            iota = jax.lax.broadcasted_iota(jnp.int16, (OUT_F, KC), 0)
            rt = jnp.where(jnp.broadcast_to(rows, (OUT_F, KC)) == iota,
                           one, zero)
            ct = jnp.where(jnp.broadcast_to(cols, (IN_F, KC)) == iota,
                           jnp.broadcast_to(vals, (IN_F, KC)), zero)
            d = jax.lax.dot_general(
                rt, ct, (((1,), (1,)), ((), ())),
                preferred_element_type=jnp.float32)
            acc = d if acc is None else acc + d
        w_ref[...] = base_ref[...] + acc.astype(jnp.bfloat16)

    slot = jax.lax.rem(i, DEPTH)
    pltpu.make_async_copy(xbufs.at[slot], xbufs.at[slot], sems.at[slot]).wait()
    xb = xbufs[slot].astype(jnp.bfloat16)
    o_ref[...] = jax.lax.dot_general(
        xb, w_ref[...], (((1,), (1,)), ((), ())),
        preferred_element_type=jnp.float32)

    @pl.when(i + DEPTH < nt)
    def _():
        nxt = pl.multiple_of((i + DEPTH) * BT, BT)
        pltpu.make_async_copy(
            x_hbm.at[pl.ds(nxt, BT), :], xbufs.at[slot], sems.at[slot]
        ).start()


def _run(xf, idx2, vals2, base16, *, interpret=False):
    t = xf.shape[0]
    return pl.pallas_call(
        _fused_kernel,
        grid=(t // BT,),
        in_specs=[
            pl.BlockSpec((1, NNZ), lambda i: (0, 0)),
            pl.BlockSpec((1, NNZ), lambda i: (0, 0)),
            pl.BlockSpec((OUT_F, IN_F), lambda i: (0, 0)),
            pl.BlockSpec(memory_space=pl.ANY),
        ],
        out_specs=pl.BlockSpec((BT, OUT_F), lambda i: (i, 0)),
        out_shape=jax.ShapeDtypeStruct((t, OUT_F), jnp.float32),
        scratch_shapes=[
            pltpu.VMEM((OUT_F, IN_F), jnp.bfloat16),
            pltpu.VMEM((DEPTH, BT, IN_F), jnp.float32),
            pltpu.SemaphoreType.DMA((DEPTH,)),
        ],
        compiler_params=pltpu.CompilerParams(
            dimension_semantics=("arbitrary",),
            vmem_limit_bytes=56 * 1024 * 1024,
        ),
        name="ortho_linear_fused",
        interpret=interpret,
    )(idx2, vals2, base16, xf)


def kernel(x, base_weight, ortho_values, ortho_indices, alpha, *, interpret=False):
    out_f, in_f = base_weight.shape
    lead = x.shape[:-1]
    xf = x.reshape(-1, in_f)

    idx2 = ortho_indices.reshape(1, NNZ)
    vals2 = (ortho_values.astype(jnp.float32)
             * alpha.astype(jnp.float32)).astype(jnp.bfloat16).reshape(1, NNZ)
    base16 = base_weight.astype(jnp.bfloat16)

    out = _run(xf, idx2, vals2, base16, interpret=interpret)
    return out.reshape(*lead, out_f)
